# baseline (speedup 1.0000x reference)
import functools

import numpy as np
import jax
import jax.numpy as jnp
from jax import lax
from jax.experimental import pallas as pl
from jax.experimental.pallas import tpu as pltpu, tpu_sc as plsc

_B, _N, _D, _R = 1, 2048, 1024, 8
_CB, _CIC, _KV = 64, 16, 64
_OP = (_D * _R) // _CB
_G = 8
_T = 512

_PERM = np.arange(_CB).reshape(8, 8).T.reshape(-1)

_NC, _NS = 2, 16
_NW = _NC * _NS
_TPW = _N // _NW
_CHT = 4
_NCH = _TPW // _CHT


def _prep(W, keys, vals):
    Wt = W.reshape(_CB * _CIC, _D)
    kp = keys.reshape(_G, 8, _KV, _CIC)
    eye = jnp.eye(8, dtype=W.dtype)
    KT = jnp.einsum('hjkg,ji->hjkig', kp, eye).reshape(_G, 8 * _KV, 8 * _CIC)
    kn = (kp ** 2).sum(-1).reshape(_G, 8 * _KV, 1)
    return Wt, KT, kn


def _kmin(sc, ko):
    m = jnp.min(sc, axis=1, keepdims=True)
    return jnp.min(jnp.where(sc == m, ko, _KV), axis=1, keepdims=True)


def _tc_body(x_ref, wa_ref, ka_ref, kna_ref, va_ref,
             wb_ref, kb_ref, knb_ref, idx_ref, t16_ref):
    xt = jnp.transpose(x_ref[...])
    pTA = jnp.dot(wa_ref[...], xt, preferred_element_type=jnp.float32)
    pTB = jnp.dot(wb_ref[...], xt, preferred_element_type=jnp.float32)
    ko = jax.lax.broadcasted_iota(jnp.int32, (8, _KV, _T), 1)
    kmA = [None] * _G
    kmB = [None] * _G
    for h in range(_G):
        pa = pTA[h * 128:(h + 1) * 128, :]
        pnA = jnp.sum((pa * pa).reshape(8, _CIC, _T), axis=1, keepdims=True)
        crossA = jnp.dot(ka_ref[h], pa, preferred_element_type=jnp.float32)
        d2A = (pnA + kna_ref[h].reshape(8, _KV, 1)) \
            - 2.0 * crossA.reshape(8, _KV, _T)
        kmA[h] = _kmin(jnp.sqrt(jnp.maximum(d2A, 0.0)), ko)
        pb = pTB[h * 128:(h + 1) * 128, :]
        pnB = jnp.sum((pb * pb).reshape(8, _CIC, _T), axis=1, keepdims=True)
        crossB = jnp.dot(kb_ref[h], pb, preferred_element_type=jnp.float32)
        d2B = (pnB + knb_ref[h].reshape(8, _KV, 1)) \
            - 2.0 * crossB.reshape(8, _KV, _T)
        kmB[h] = _kmin(jnp.sqrt(jnp.maximum(d2B, 0.0)), ko)
    t = None
    for q in range(_G):
        kAq = jnp.concatenate([kmA[r][q:q + 1] for r in range(8)], axis=0)
        PT = jnp.dot(va_ref[q], xt[q * 128:(q + 1) * 128, :],
                     preferred_element_type=jnp.float32).reshape(8, _KV, _T)
        s = jnp.sum(jnp.where(ko == kAq, PT, 0.0), axis=1, keepdims=True)
        t = s if t is None else t + s
    km_all = jnp.concatenate([k.reshape(8, _T) for k in kmB], axis=0)
    c64 = jax.lax.broadcasted_iota(jnp.int32, (_CB, _T), 0)
    idx_ref[...] = jnp.transpose(km_all + c64 * _KV)
    tT = jnp.transpose(t.reshape(8, _T))
    t16_ref[...] = jnp.concatenate(
        [jnp.broadcast_to(tT[:, j:j + 1], (_T, 16)) for j in range(8)],
        axis=1)


def _tc_stage(x, WAt, KAT, knA, VA, WBt, KBT, knB):
    full = lambda *s: pl.BlockSpec(s, lambda i: (0,) * len(s))
    return pl.pallas_call(
        _tc_body,
        grid=(_N // _T,),
        in_specs=[
            pl.BlockSpec((_T, _D), lambda i: (i, 0)),
            full(_CB * _CIC, _D),
            full(_G, 8 * _KV, 8 * _CIC),
            full(_G, 8 * _KV, 1),
            full(_G, 8 * _KV, _OP),
            full(_CB * _CIC, _D),
            full(_G, 8 * _KV, 8 * _CIC),
            full(_G, 8 * _KV, 1),
        ],
        out_specs=[pl.BlockSpec((_T, _CB), lambda i: (i, 0)),
                   pl.BlockSpec((_T, _OP), lambda i: (i, 0))],
        out_shape=[jax.ShapeDtypeStruct((_N, _CB), jnp.int32),
                   jax.ShapeDtypeStruct((_N, _OP), jnp.float32)],
    )(x, WAt, KAT, knA, VA, WBt, KBT, knB)


def _sc_body(vb_ref, idx_ref, t16_ref, out_ref,
             idx_v, t_v, buf0, buf1, ov0, ov1, sem0, sem1, osem):
    wid = lax.axis_index("s") * _NC + lax.axis_index("c")
    base = wid * _TPW
    pltpu.sync_copy(idx_ref.at[pl.ds(base * _CB, _TPW * _CB)], idx_v)
    pltpu.sync_copy(t16_ref.at[pl.ds(base, _TPW)], t_v)
    bufs = (buf0, buf1)
    sems = (sem0, sem1)
    ovs = (ov0, ov1)

    def _fire(ch, b):
        cc = jnp.minimum(ch, _NCH - 1)
        for half in range(_CHT // 2):
            pltpu.async_copy(
                vb_ref.at[idx_v.at[pl.ds((cc * _CHT + 2 * half) * _CB,
                                         2 * _CB)]],
                bufs[b].at[pl.ds(half * 2 * _CB, 2 * _CB)],
                sems[b])

    def _tok(ch, k, buf):
        n = _CHT * ch + k
        ov = ovs[k % 2]
        tvs = [t_v[n, pl.ds(r * 16, 16)] for r in range(8)]
        for q in range(8):
            for v in range(8):
                acc = None
                for r in range(8):
                    rv = buf[k * _CB + 8 * r + q, pl.ds(v * 16, 16)]
                    p = tvs[r] * rv
                    acc = p if acc is None else acc + p
                ov[pl.ds(q * 128 + v * 16, 16)] = acc
        pltpu.async_copy(ov, out_ref.at[base + n], osem)

    _fire(0, 0)
    _fire(1, 1)

    @pl.loop(0, _NCH, step=2)
    def _chunks(i):
        for b in range(2):
            ch = i + b
            pltpu.make_async_copy(
                vb_ref.at[idx_v.at[pl.ds(0, 2 * _CB)]], bufs[b],
                sems[b]).wait()
            for k in range(_CHT):
                if k >= 2:
                    pltpu.make_async_copy(ov0, out_ref.at[base],
                                          osem).wait()
                else:
                    @pl.when(ch > 0)
                    def _():
                        pltpu.make_async_copy(ov0, out_ref.at[base],
                                              osem).wait()
                _tok(ch, k, bufs[b])
            _fire(ch + 2, b)

    pltpu.make_async_copy(
        vb_ref.at[idx_v.at[pl.ds(0, 2 * _CB)]], buf0, sem0).wait()
    pltpu.make_async_copy(
        vb_ref.at[idx_v.at[pl.ds(0, 2 * _CB)]], buf1, sem1).wait()
    pltpu.make_async_copy(ov0, out_ref.at[base], osem).wait()
    pltpu.make_async_copy(ov1, out_ref.at[base], osem).wait()


@functools.cache
def _sc_combine():
    return pl.kernel(
        _sc_body,
        mesh=plsc.VectorSubcoreMesh(core_axis_name="c", subcore_axis_name="s",
                                    num_cores=_NC, num_subcores=_NS),
        out_type=jax.ShapeDtypeStruct((_N, _D), jnp.float32),
        scratch_types=[
            pltpu.VMEM((_TPW * _CB,), jnp.int32),
            pltpu.VMEM((_TPW, _OP), jnp.float32),
            pltpu.VMEM((_CHT * _CB, _OP), jnp.float32),
            pltpu.VMEM((_CHT * _CB, _OP), jnp.float32),
            pltpu.VMEM((_D,), jnp.float32),
            pltpu.VMEM((_D,), jnp.float32),
            pltpu.SemaphoreType.DMA,
            pltpu.SemaphoreType.DMA,
            pltpu.SemaphoreType.DMA,
        ],
    )


@jax.jit
def _run(x, W_A, keys_A, vals_A, W_B, keys_B, vals_B):
    WAt, KAT, knA = _prep(W_A, keys_A, vals_A)
    WBt, KBT, knB = _prep(W_B, keys_B, vals_B)
    VA = vals_A[_PERM].reshape(_G, 8 * _KV, _OP)
    idx, t16 = _tc_stage(x.reshape(_N, _D), WAt, KAT, knA, VA,
                         WBt, KBT, knB)
    vb = vals_B.reshape(_CB * _KV, _OP)
    out = _sc_combine()(vb, idx.reshape(_N * _CB), t16)
    return out.reshape(_B, _N, _D)


def kernel(x, W_A, keys_A, vals_A, W_B, keys_B, vals_B):
    return _run(x, W_A, keys_A, vals_A, W_B, keys_B, vals_B)


# speedup vs baseline: 1.0757x; 1.0757x over previous
import functools

import numpy as np
import jax
import jax.numpy as jnp
from jax import lax
from jax.experimental import pallas as pl
from jax.experimental.pallas import tpu as pltpu, tpu_sc as plsc

_B, _N, _D, _R = 1, 2048, 1024, 8
_CB, _CIC, _KV = 64, 16, 64
_OP = (_D * _R) // _CB
_G = 8
_T = 512

_PERM = np.arange(_CB).reshape(8, 8).T.reshape(-1)

_NC, _NS = 2, 16
_NW = _NC * _NS
_TPW = _N // _NW
_NCH = _TPW // 2


def _prep(W, keys, vals):
    Wt = W.reshape(_CB * _CIC, _D)
    kp = keys.reshape(_G, 8, _KV, _CIC)
    eye = jnp.eye(8, dtype=W.dtype)
    KT = jnp.einsum('hjkg,ji->hjkig', kp, eye).reshape(_G, 8 * _KV, 8 * _CIC)
    kn = (kp ** 2).sum(-1).reshape(_G, 8 * _KV, 1)
    return Wt, KT, kn


def _kmin(sc, ko):
    m = jnp.min(sc, axis=1, keepdims=True)
    return jnp.min(jnp.where(sc == m, ko, _KV), axis=1, keepdims=True)


def _tc_body(x_ref, wa_ref, ka_ref, kna_ref, va_ref,
             wb_ref, kb_ref, knb_ref, idx_ref, t16_ref):
    xt = jnp.transpose(x_ref[...])
    pTA = jnp.dot(wa_ref[...], xt, preferred_element_type=jnp.float32)
    pTB = jnp.dot(wb_ref[...], xt, preferred_element_type=jnp.float32)
    ko = jax.lax.broadcasted_iota(jnp.int32, (8, _KV, _T), 1)
    kmA = [None] * _G
    kmB = [None] * _G
    for h in range(_G):
        pa = pTA[h * 128:(h + 1) * 128, :]
        pnA = jnp.sum((pa * pa).reshape(8, _CIC, _T), axis=1, keepdims=True)
        crossA = jnp.dot(ka_ref[h], pa, preferred_element_type=jnp.float32)
        d2A = (pnA + kna_ref[h].reshape(8, _KV, 1)) \
            - 2.0 * crossA.reshape(8, _KV, _T)
        kmA[h] = _kmin(jnp.sqrt(jnp.maximum(d2A, 0.0)), ko)
        pb = pTB[h * 128:(h + 1) * 128, :]
        pnB = jnp.sum((pb * pb).reshape(8, _CIC, _T), axis=1, keepdims=True)
        crossB = jnp.dot(kb_ref[h], pb, preferred_element_type=jnp.float32)
        d2B = (pnB + knb_ref[h].reshape(8, _KV, 1)) \
            - 2.0 * crossB.reshape(8, _KV, _T)
        kmB[h] = _kmin(jnp.sqrt(jnp.maximum(d2B, 0.0)), ko)
    t = None
    for q in range(_G):
        kAq = jnp.concatenate([kmA[r][q:q + 1] for r in range(8)], axis=0)
        PT = jnp.dot(va_ref[q], xt[q * 128:(q + 1) * 128, :],
                     preferred_element_type=jnp.float32).reshape(8, _KV, _T)
        s = jnp.sum(jnp.where(ko == kAq, PT, 0.0), axis=1, keepdims=True)
        t = s if t is None else t + s
    km_all = jnp.concatenate([k.reshape(8, _T) for k in kmB], axis=0)
    c64 = jax.lax.broadcasted_iota(jnp.int32, (_CB, _T), 0)
    idx_ref[...] = jnp.transpose(km_all + c64 * _KV)
    tT = jnp.transpose(t.reshape(8, _T))
    t16_ref[...] = jnp.concatenate(
        [jnp.broadcast_to(tT[:, j:j + 1], (_T, 16)) for j in range(8)],
        axis=1)


def _tc_stage(x, WAt, KAT, knA, VA, WBt, KBT, knB):
    full = lambda *s: pl.BlockSpec(s, lambda i: (0,) * len(s))
    return pl.pallas_call(
        _tc_body,
        grid=(_N // _T,),
        in_specs=[
            pl.BlockSpec((_T, _D), lambda i: (i, 0)),
            full(_CB * _CIC, _D),
            full(_G, 8 * _KV, 8 * _CIC),
            full(_G, 8 * _KV, 1),
            full(_G, 8 * _KV, _OP),
            full(_CB * _CIC, _D),
            full(_G, 8 * _KV, 8 * _CIC),
            full(_G, 8 * _KV, 1),
        ],
        out_specs=[pl.BlockSpec((_T, _CB), lambda i: (i, 0)),
                   pl.BlockSpec((_T, _OP), lambda i: (i, 0))],
        out_shape=[jax.ShapeDtypeStruct((_N, _CB), jnp.int32),
                   jax.ShapeDtypeStruct((_N, _OP), jnp.float32)],
    )(x, WAt, KAT, knA, VA, WBt, KBT, knB)


def _sc_body(vb_ref, idx_ref, t16_ref, out_ref,
             idx_v, t_v, buf0, buf1, ov0, ov1, sem0, sem1, osem):
    wid = lax.axis_index("s") * _NC + lax.axis_index("c")
    base = wid * _TPW
    pltpu.sync_copy(idx_ref.at[pl.ds(base * _CB, _TPW * _CB)], idx_v)
    pltpu.sync_copy(t16_ref.at[pl.ds(base, _TPW)], t_v)
    bufs = (buf0, buf1)
    sems = (sem0, sem1)
    ovs = (ov0, ov1)

    def _fire(ch, b):
        cc = jnp.minimum(ch, _NCH - 1)
        pltpu.async_copy(
            vb_ref.at[idx_v.at[pl.ds(cc * 2 * _CB, 2 * _CB)]], bufs[b],
            sems[b])

    def _tok(ch, k, buf):
        n = 2 * ch + k
        ov = ovs[k]
        tvs = [t_v[n, pl.ds(r * 16, 16)] for r in range(8)]
        for q in range(8):
            for v in range(8):
                acc = None
                for r in range(8):
                    rv = buf[k * _CB + 8 * r + q, pl.ds(v * 16, 16)]
                    p = tvs[r] * rv
                    acc = p if acc is None else acc + p
                ov[pl.ds(q * 128 + v * 16, 16)] = acc
        pltpu.async_copy(ov, out_ref.at[base + n], osem)

    _fire(0, 0)
    _fire(1, 1)

    @pl.loop(0, _NCH, step=2)
    def _chunks(i):
        for b in range(2):
            ch = i + b
            pltpu.make_async_copy(
                vb_ref.at[idx_v.at[pl.ds(0, 2 * _CB)]], bufs[b],
                sems[b]).wait()
            @pl.when(ch > 0)
            def _():
                pltpu.make_async_copy(ov0, out_ref.at[base], osem).wait()
                pltpu.make_async_copy(ov1, out_ref.at[base], osem).wait()
            for k in range(2):
                _tok(ch, k, bufs[b])
            _fire(ch + 2, b)

    pltpu.make_async_copy(
        vb_ref.at[idx_v.at[pl.ds(0, 2 * _CB)]], buf0, sem0).wait()
    pltpu.make_async_copy(
        vb_ref.at[idx_v.at[pl.ds(0, 2 * _CB)]], buf1, sem1).wait()
    pltpu.make_async_copy(ov0, out_ref.at[base], osem).wait()
    pltpu.make_async_copy(ov1, out_ref.at[base], osem).wait()


@functools.cache
def _sc_combine():
    return pl.kernel(
        _sc_body,
        mesh=plsc.VectorSubcoreMesh(core_axis_name="c", subcore_axis_name="s",
                                    num_cores=_NC, num_subcores=_NS),
        out_type=jax.ShapeDtypeStruct((_N, _D), jnp.float32),
        scratch_types=[
            pltpu.VMEM((_TPW * _CB,), jnp.int32),
            pltpu.VMEM((_TPW, _OP), jnp.float32),
            pltpu.VMEM((2 * _CB, _OP), jnp.float32),
            pltpu.VMEM((2 * _CB, _OP), jnp.float32),
            pltpu.VMEM((_D,), jnp.float32),
            pltpu.VMEM((_D,), jnp.float32),
            pltpu.SemaphoreType.DMA,
            pltpu.SemaphoreType.DMA,
            pltpu.SemaphoreType.DMA,
        ],
    )


@jax.jit
def _run(x, W_A, keys_A, vals_A, W_B, keys_B, vals_B):
    WAt, KAT, knA = _prep(W_A, keys_A, vals_A)
    WBt, KBT, knB = _prep(W_B, keys_B, vals_B)
    VA = vals_A[_PERM].reshape(_G, 8 * _KV, _OP)
    idx, t16 = _tc_stage(x.reshape(_N, _D), WAt, KAT, knA, VA,
                         WBt, KBT, knB)
    vb = vals_B.reshape(_CB * _KV, _OP)
    out = _sc_combine()(vb, idx.reshape(_N * _CB), t16)
    return out.reshape(_B, _N, _D)


def kernel(x, W_A, keys_A, vals_A, W_B, keys_B, vals_B):
    return _run(x, W_A, keys_A, vals_A, W_B, keys_B, vals_B)
